# Initial kernel scaffold; baseline (speedup 1.0000x reference)
import jax
import jax.numpy as jnp
from jax.experimental import pallas as pl


def kernel(local_feats, W1, b1, W2, b2):
    raise NotImplementedError("write your pallas kernel here")


# speedup vs baseline: 1.4730x; 1.4730x over previous
import jax
import jax.numpy as jnp
from jax import lax
from jax.experimental import pallas as pl

_B, _N, _D, _H = 4, 8192, 768, 4
_HID = _D // 4
_K = 256
_HP = 8
_BN = 512
_BNC = 1024


def _scores_body(x_ref, w1_ref, b1_ref, w2_ref, b2_ref, out_ref):
    x = x_ref[0]
    h = jnp.dot(x, w1_ref[...], preferred_element_type=jnp.float32)
    h = jnp.maximum(h + b1_ref[...], 0.0)
    st = lax.dot_general(w2_ref[...], h, (((1,), (1,)), ((), ())),
                         preferred_element_type=jnp.float32)
    out_ref[0] = st + b2_ref[...]


def _select_body(s_ref, mask_ref, div_ref):
    S = s_ref[...]
    i = lax.bitcast_convert_type(S, jnp.int32)
    key = i ^ ((i >> 31) & jnp.int32(0x7FFFFFFF))

    kf = jnp.float32(_K)
    cnt0 = jnp.sum((key >= 0).astype(jnp.float32), axis=1, keepdims=True)
    prefix = jnp.where(cnt0 >= kf, jnp.int32(0), jnp.int32(-(2 ** 31)))
    for t in range(31):
        cand = prefix + jnp.int32(1 << (30 - t))
        cnt = jnp.sum((key >= cand).astype(jnp.float32), axis=1, keepdims=True)
        prefix = jnp.where(cnt >= kf, cand, prefix)
    T = prefix

    gt = (key > T)
    need = kf - jnp.sum(gt.astype(jnp.float32), axis=1, keepdims=True)

    idx = lax.broadcasted_iota(jnp.int32, (16, _N), 1)
    key2 = jnp.where(key == T, jnp.int32(_N) - idx, jnp.int32(0))
    prefix2 = jnp.zeros((16, 1), dtype=jnp.int32)
    for t in range(14):
        cand = prefix2 + jnp.int32(1 << (13 - t))
        cnt = jnp.sum((key2 >= cand).astype(jnp.float32), axis=1, keepdims=True)
        prefix2 = jnp.where(cnt >= need, cand, prefix2)
    T2 = prefix2

    sel = jnp.logical_or(gt, key2 >= T2).astype(jnp.float32)

    r4 = lax.broadcasted_iota(jnp.int32, (4, 16), 0)
    c4 = lax.broadcasted_iota(jnp.int32, (4, 16), 1)
    bm4 = (c4 // _H == r4).astype(jnp.float32)
    mask_ref[...] = jnp.minimum(
        lax.dot_general(bm4, sel, (((1,), (0,)), ((), ())),
                        preferred_element_type=jnp.float32,
                        precision=lax.Precision.HIGHEST), 1.0)

    ss = jnp.sum(S * S, axis=1, keepdims=True)
    inv = 1.0 / jnp.maximum(jnp.sqrt(ss), 1e-6)
    Sn = S * inv
    G = lax.dot_general(Sn, Sn, (((1,), (1,)), ((), ())),
                        preferred_element_type=jnp.float32,
                        precision=lax.Precision.HIGHEST)
    r = lax.broadcasted_iota(jnp.int32, (16, 16), 0)
    c = lax.broadcasted_iota(jnp.int32, (16, 16), 1)
    eye = (r == c).astype(jnp.float32)
    blk = (r // _H == c // _H).astype(jnp.float32)
    div_ref[...] = jnp.sum(jnp.abs(G - eye) * blk,
                           keepdims=True).reshape(1, 1) / jnp.float32(_B * _H * _H)


def _apply_body(x_ref, m_ref, out_ref):
    out_ref[0] = x_ref[0] * m_ref[0]


def kernel(local_feats, W1, b1, W2, b2):
    x = local_feats
    w1r = jnp.transpose(W1, (1, 0, 2)).reshape(_D, _H * _HID)
    b1r = b1.reshape(1, _H * _HID)
    w2pad = jnp.zeros((_HP, _D), dtype=jnp.float32)
    for h in range(_H):
        w2pad = w2pad.at[h, h * _HID:(h + 1) * _HID].set(W2[h, :, 0])
    b2pad = jnp.zeros((_HP, 1), dtype=jnp.float32)
    b2pad = b2pad.at[:_H, 0].set(b2[:, 0])

    scores_t = pl.pallas_call(
        _scores_body,
        grid=(_B, _N // _BN),
        in_specs=[
            pl.BlockSpec((1, _BN, _D), lambda b, n: (b, n, 0)),
            pl.BlockSpec((_D, _H * _HID), lambda b, n: (0, 0)),
            pl.BlockSpec((1, _H * _HID), lambda b, n: (0, 0)),
            pl.BlockSpec((_HP, _D), lambda b, n: (0, 0)),
            pl.BlockSpec((_HP, 1), lambda b, n: (0, 0)),
        ],
        out_specs=pl.BlockSpec((1, _HP, _BN), lambda b, n: (b, 0, n)),
        out_shape=jax.ShapeDtypeStruct((_B, _HP, _N), jnp.float32),
    )(x, w1r, b1r, w2pad, b2pad)

    S = scores_t[:, :_H, :].reshape(_B * _H, _N)

    mask2d, div = pl.pallas_call(
        _select_body,
        out_shape=[
            jax.ShapeDtypeStruct((_B, _N), jnp.float32),
            jax.ShapeDtypeStruct((1, 1), jnp.float32),
        ],
    )(S)

    ste_mask = mask2d.reshape(_B, _N, 1)

    selected = pl.pallas_call(
        _apply_body,
        grid=(_B, _N // _BNC),
        in_specs=[
            pl.BlockSpec((1, _BNC, _D), lambda b, n: (b, n, 0)),
            pl.BlockSpec((1, _BNC, 1), lambda b, n: (b, n, 0)),
        ],
        out_specs=pl.BlockSpec((1, _BNC, _D), lambda b, n: (b, n, 0)),
        out_shape=jax.ShapeDtypeStruct((_B, _N, _D), jnp.float32),
    )(x, ste_mask)

    return selected, div.reshape(()), ste_mask


# speedup vs baseline: 1.5954x; 1.0831x over previous
import jax
import jax.numpy as jnp
from jax import lax
from jax.experimental import pallas as pl

_B, _N, _D, _H = 4, 8192, 768, 4
_HID = _D // 4
_K = 256
_HP = 8
_BN = 1024
_BNC = 1024


def _scores_body(x_ref, w1_ref, b1_ref, w2_ref, b2_ref, out_ref):
    x = x_ref[0]
    h = jnp.dot(x, w1_ref[...], preferred_element_type=jnp.float32)
    h = jnp.maximum(h + b1_ref[...], 0.0)
    st = lax.dot_general(w2_ref[...], h, (((1,), (1,)), ((), ())),
                         preferred_element_type=jnp.float32)
    out_ref[0] = st + b2_ref[...]


def _select_body(s_ref, mask_ref, div_ref):
    S = s_ref[...]
    kf = jnp.float32(_K)
    _G = 8
    idx = lax.broadcasted_iota(jnp.int32, (_G, _N), 1)

    for g in range(2):
        i = lax.bitcast_convert_type(S[g * _G:(g + 1) * _G, :], jnp.int32)
        key = i ^ ((i >> 31) & jnp.int32(0x7FFFFFFF))

        cnt0 = jnp.sum((key >= 0).astype(jnp.float32), axis=1, keepdims=True)
        prefix = jnp.where(cnt0 >= kf, jnp.int32(0), jnp.int32(-(2 ** 31)))
        for t in range(31):
            cand = prefix + jnp.int32(1 << (30 - t))
            cnt = jnp.sum((key >= cand).astype(jnp.float32), axis=1,
                          keepdims=True)
            prefix = jnp.where(cnt >= kf, cand, prefix)
        T = prefix

        gt = (key > T)
        need = kf - jnp.sum(gt.astype(jnp.float32), axis=1, keepdims=True)

        key2 = jnp.where(key == T, jnp.int32(_N) - idx, jnp.int32(0))
        prefix2 = jnp.zeros((_G, 1), dtype=jnp.int32)
        for t in range(14):
            cand = prefix2 + jnp.int32(1 << (13 - t))
            cnt = jnp.sum((key2 >= cand).astype(jnp.float32), axis=1,
                          keepdims=True)
            prefix2 = jnp.where(cnt >= need, cand, prefix2)

        sel = jnp.logical_or(gt, key2 >= prefix2).astype(jnp.float32)
        m0 = jnp.minimum(jnp.sum(sel[0:_H, :], axis=0, keepdims=True), 1.0)
        m1 = jnp.minimum(jnp.sum(sel[_H:_G, :], axis=0, keepdims=True), 1.0)
        mask_ref[2 * g:2 * g + 1, :] = m0
        mask_ref[2 * g + 1:2 * g + 2, :] = m1

    ss = jnp.sum(S * S, axis=1, keepdims=True)
    inv = 1.0 / jnp.maximum(jnp.sqrt(ss), 1e-6)
    Sn = S * inv
    G = lax.dot_general(Sn, Sn, (((1,), (1,)), ((), ())),
                        preferred_element_type=jnp.float32,
                        precision=lax.Precision.HIGHEST)
    r = lax.broadcasted_iota(jnp.int32, (16, 16), 0)
    c = lax.broadcasted_iota(jnp.int32, (16, 16), 1)
    eye = (r == c).astype(jnp.float32)
    blk = (r // _H == c // _H).astype(jnp.float32)
    div_ref[...] = jnp.sum(jnp.abs(G - eye) * blk,
                           keepdims=True).reshape(1, 1) / jnp.float32(_B * _H * _H)


def _apply_body(x_ref, m_ref, out_ref):
    out_ref[0] = x_ref[0] * m_ref[0]


def kernel(local_feats, W1, b1, W2, b2):
    x = local_feats
    w1r = jnp.transpose(W1, (1, 0, 2)).reshape(_D, _H * _HID)
    b1r = b1.reshape(1, _H * _HID)
    w2pad = jnp.zeros((_HP, _D), dtype=jnp.float32)
    for h in range(_H):
        w2pad = w2pad.at[h, h * _HID:(h + 1) * _HID].set(W2[h, :, 0])
    b2pad = jnp.zeros((_HP, 1), dtype=jnp.float32)
    b2pad = b2pad.at[:_H, 0].set(b2[:, 0])

    scores_t = pl.pallas_call(
        _scores_body,
        grid=(_B, _N // _BN),
        in_specs=[
            pl.BlockSpec((1, _BN, _D), lambda b, n: (b, n, 0)),
            pl.BlockSpec((_D, _H * _HID), lambda b, n: (0, 0)),
            pl.BlockSpec((1, _H * _HID), lambda b, n: (0, 0)),
            pl.BlockSpec((_HP, _D), lambda b, n: (0, 0)),
            pl.BlockSpec((_HP, 1), lambda b, n: (0, 0)),
        ],
        out_specs=pl.BlockSpec((1, _HP, _BN), lambda b, n: (b, 0, n)),
        out_shape=jax.ShapeDtypeStruct((_B, _HP, _N), jnp.float32),
    )(x, w1r, b1r, w2pad, b2pad)

    S = scores_t[:, :_H, :].reshape(_B * _H, _N)

    mask2d, div = pl.pallas_call(
        _select_body,
        out_shape=[
            jax.ShapeDtypeStruct((_B, _N), jnp.float32),
            jax.ShapeDtypeStruct((1, 1), jnp.float32),
        ],
    )(S)

    ste_mask = mask2d.reshape(_B, _N, 1)

    selected = pl.pallas_call(
        _apply_body,
        grid=(_B, _N // _BNC),
        in_specs=[
            pl.BlockSpec((1, _BNC, _D), lambda b, n: (b, n, 0)),
            pl.BlockSpec((1, _BNC, 1), lambda b, n: (b, n, 0)),
        ],
        out_specs=pl.BlockSpec((1, _BNC, _D), lambda b, n: (b, n, 0)),
        out_shape=jax.ShapeDtypeStruct((_B, _N, _D), jnp.float32),
    )(x, ste_mask)

    return selected, div.reshape(()), ste_mask
